# Initial kernel scaffold; baseline (speedup 1.0000x reference)
import jax
import jax.numpy as jnp
from jax.experimental import pallas as pl


def kernel(x, weight):
    raise NotImplementedError("write your pallas kernel here")


# speedup vs baseline: 1.6838x; 1.6838x over previous
import functools

import jax
import jax.numpy as jnp
from jax import lax
from jax.experimental import pallas as pl
from jax.experimental.pallas import tpu as pltpu
from jax.experimental.pallas import tpu_sc as plsc

_CH = 128


def kernel(x, weight):
    B, H = x.shape
    V, D = weight.shape
    info = plsc.get_sparse_core_info()
    nw = info.num_cores * info.num_subcores
    tot = B * H
    per_w = tot // nw
    nch = per_w // _CH
    assert tot == nw * nch * _CH, (tot, nw, nch)

    idx = x.reshape(nw * nch, _CH).astype(jnp.int32)
    mesh = plsc.VectorSubcoreMesh(core_axis_name="c", subcore_axis_name="s")

    @functools.partial(
        pl.kernel,
        mesh=mesh,
        out_type=jax.ShapeDtypeStruct((tot, D), jnp.float32),
        scratch_types=[
            pltpu.VMEM((nch, _CH), jnp.int32),
            pltpu.VMEM((_CH, D), jnp.float32),
            pltpu.SemaphoreType.DMA,
        ],
        compiler_params=pltpu.CompilerParams(use_tc_tiling_on_sc=False),
    )
    def run(idx_hbm, w_hbm, out_hbm, idx_v, rows_v, sem):
        wid = lax.axis_index("s") * info.num_cores + lax.axis_index("c")
        pltpu.sync_copy(idx_hbm.at[pl.ds(wid * nch, nch)], idx_v)

        def chunk(c, carry):
            pltpu.async_copy(w_hbm.at[idx_v.at[c]], rows_v, sem).wait()
            pltpu.sync_copy(rows_v, out_hbm.at[pl.ds((wid * nch + c) * _CH, _CH)])
            return carry

        lax.fori_loop(0, nch, chunk, 0)

    out = run(idx, weight)
    return out.reshape(B, H, D)


# speedup vs baseline: 1.8742x; 1.1131x over previous
import functools

import jax
import jax.numpy as jnp
from jax import lax
from jax.experimental import pallas as pl
from jax.experimental.pallas import tpu as pltpu
from jax.experimental.pallas import tpu_sc as plsc

_CH = 128
_G = 4


def kernel(x, weight):
    B, H = x.shape
    V, D = weight.shape
    info = plsc.get_sparse_core_info()
    nw = info.num_cores * info.num_subcores
    tot = B * H
    per_w = tot // nw
    nch = per_w // _CH
    ng = nch // _G
    rows_g = _G * _CH
    assert tot == nw * ng * rows_g and ng % 2 == 0, (tot, nw, nch, ng)

    idx = x.reshape(nw * nch, _CH).astype(jnp.int32)
    mesh = plsc.VectorSubcoreMesh(core_axis_name="c", subcore_axis_name="s")

    @functools.partial(
        pl.kernel,
        mesh=mesh,
        out_type=jax.ShapeDtypeStruct((tot, D), jnp.float32),
        scratch_types=[
            pltpu.VMEM((nch, _CH), jnp.int32),
            pltpu.VMEM((2, rows_g, D), jnp.float32),
            pltpu.SemaphoreType.DMA,
            pltpu.SemaphoreType.DMA,
            pltpu.SemaphoreType.DMA,
            pltpu.SemaphoreType.DMA,
        ],
        compiler_params=pltpu.CompilerParams(use_tc_tiling_on_sc=False),
    )
    def run(idx_hbm, w_hbm, out_hbm, idx_v, rows_v, gsem0, gsem1, osem0, osem1):
        wid = lax.axis_index("s") * info.num_cores + lax.axis_index("c")
        base = wid * per_w
        gsems = (gsem0, gsem1)
        osems = (osem0, osem1)
        pltpu.sync_copy(idx_hbm.at[pl.ds(wid * nch, nch)], idx_v)

        def issue_gathers(g, bank):
            for j in range(_G):
                c = g * _G + j
                pltpu.async_copy(
                    w_hbm.at[idx_v.at[c]],
                    rows_v.at[bank].at[pl.ds(j * _CH, _CH)],
                    gsems[bank],
                )

        def wait_gathers(bank):
            pltpu.make_async_copy(
                out_hbm.at[pl.ds(0, rows_g)], rows_v.at[bank], gsems[bank]
            ).wait()

        def issue_out(g, bank):
            pltpu.async_copy(
                rows_v.at[bank],
                out_hbm.at[pl.ds(base + g * rows_g, rows_g)],
                osems[bank],
            )

        def wait_out(bank):
            pltpu.make_async_copy(
                rows_v.at[bank], out_hbm.at[pl.ds(base, rows_g)], osems[bank]
            ).wait()

        issue_gathers(0, 0)
        issue_gathers(1, 1)
        wait_gathers(0)
        issue_out(0, 0)

        def pair(p, carry):
            for b, g in ((1, 2 * p + 1), (0, 2 * p + 2)):
                wait_out(1 - b)
                issue_gathers(g + 1, 1 - b)
                wait_gathers(b)
                issue_out(g, b)
            return carry

        lax.fori_loop(0, (ng - 2) // 2, pair, 0)

        wait_out(0)
        wait_gathers(1)
        issue_out(ng - 1, 1)
        wait_out(1)

    out = run(idx, weight)
    return out.reshape(B, H, D)
